# baseline (speedup 1.0000x reference)
import functools

import jax
import jax.numpy as jnp
from jax import lax
from jax.experimental import pallas as pl
from jax.experimental.pallas import tpu as pltpu
from jax.experimental.pallas import tpu_sc as plsc

B = 16384
EMB = 32


def _make_sc_gather():
    info = plsc.get_sparse_core_info()
    nw = info.num_cores * info.num_subcores
    bpw = B // nw
    mesh = plsc.VectorSubcoreMesh(core_axis_name="c", subcore_axis_name="s")

    @functools.partial(
        pl.kernel,
        out_type=[
            jax.ShapeDtypeStruct((B, EMB), jnp.float32),
            jax.ShapeDtypeStruct((B, EMB), jnp.float32),
        ],
        mesh=mesh,
        compiler_params=pltpu.CompilerParams(use_tc_tiling_on_sc=False),
        scratch_types=[
            pltpu.VMEM((bpw,), jnp.int32),
            pltpu.VMEM((bpw, EMB), jnp.float32),
            pltpu.VMEM((bpw,), jnp.int32),
            pltpu.VMEM((bpw, EMB), jnp.float32),
            pltpu.SemaphoreType.DMA,
            pltpu.SemaphoreType.DMA,
        ],
    )
    def sc_gather(uidx_hbm, iidx_hbm, utab_hbm, itab_hbm, uout_hbm, iout_hbm,
                  uidx_v, urows_v, iidx_v, irows_v, usem, isem):
        wid = lax.axis_index("s") * info.num_cores + lax.axis_index("c")
        base = wid * bpw
        pltpu.sync_copy(uidx_hbm.at[pl.ds(base, bpw)], uidx_v)
        pltpu.sync_copy(iidx_hbm.at[pl.ds(base, bpw)], iidx_v)
        cu = pltpu.async_copy(utab_hbm.at[uidx_v], urows_v, usem)
        ci = pltpu.async_copy(itab_hbm.at[iidx_v], irows_v, isem)
        cu.wait()
        ci.wait()
        pltpu.sync_copy(urows_v, uout_hbm.at[pl.ds(base, bpw)])
        pltpu.sync_copy(irows_v, iout_hbm.at[pl.ds(base, bpw)])

    return sc_gather


_sc_gather = _make_sc_gather()


_BLK = 2048


def _mlp_body(u_ref, i_ref, w1a_ref, w1b_ref, b1_ref, w2_ref, b2_ref,
              w3_ref, b3_ref, wo_ref, bo_ref, out_ref):
    h = u_ref[...] @ w1a_ref[...] + i_ref[...] @ w1b_ref[...] + b1_ref[...]
    h = jnp.maximum(h, 0.0)
    h = jnp.maximum(h @ w2_ref[...] + b2_ref[...], 0.0)
    h = jnp.maximum(h @ w3_ref[...] + b3_ref[...], 0.0)
    out_ref[...] = h @ wo_ref[...] + bo_ref[...]


def _mlp(u, i, w1a, w1b, b1, w2t, b2, w3t, b3, wot, bo):
    grid = (B // _BLK,)
    full = lambda g: (0, 0)
    return pl.pallas_call(
        _mlp_body,
        grid=grid,
        in_specs=[
            pl.BlockSpec((_BLK, EMB), lambda g: (g, 0)),
            pl.BlockSpec((_BLK, EMB), lambda g: (g, 0)),
            pl.BlockSpec(w1a.shape, full),
            pl.BlockSpec(w1b.shape, full),
            pl.BlockSpec(b1.shape, full),
            pl.BlockSpec(w2t.shape, full),
            pl.BlockSpec(b2.shape, full),
            pl.BlockSpec(w3t.shape, full),
            pl.BlockSpec(b3.shape, full),
            pl.BlockSpec(wot.shape, full),
            pl.BlockSpec(bo.shape, full),
        ],
        out_specs=pl.BlockSpec((_BLK, 1), lambda g: (g, 0)),
        out_shape=jax.ShapeDtypeStruct((B, 1), jnp.float32),
    )(u, i, w1a, w1b, b1, w2t, b2, w3t, b3, wot, bo)


@jax.jit
def kernel(user_indices, item_indices, user_table, item_table,
           W1, b1, W2, b2, W3, b3, Wo, bo):
    uidx = user_indices.astype(jnp.int32)
    iidx = item_indices.astype(jnp.int32)
    u, i = _sc_gather(uidx, iidx, user_table, item_table)
    w1a = W1[:, :EMB].T
    w1b = W1[:, EMB:].T
    out = _mlp(u, i, w1a, w1b, b1.reshape(1, -1), W2.T, b2.reshape(1, -1),
               W3.T, b3.reshape(1, -1), Wo.T, bo.reshape(1, -1))
    return out.reshape(B)
